# Initial kernel scaffold; baseline (speedup 1.0000x reference)
import jax
import jax.numpy as jnp
from jax.experimental import pallas as pl


def kernel(x, t, edge_index, edge_attr, batch_size, em_w1, em_b1, em_w2, em_b2, em_w3, em_b3, ee_w1, ee_b1, ee_w2, ee_b2, ee_w3, ee_b3, nm_w1, nm_b1, nm_w2, nm_b2, nm_w3, nm_b3):
    raise NotImplementedError("write your pallas kernel here")


# speedup vs baseline: 2.7893x; 2.7893x over previous
import functools

import jax
import jax.numpy as jnp
from jax import lax
from jax.experimental import pallas as pl
from jax.experimental.pallas import tpu as pltpu
from jax.experimental.pallas import tpu_sc as plsc

N = 10000
E = 320000
D = 128
ED = 16
H = 128

NC = 2
NS = 16
W = NC * NS
EW = E // W
C = 80
NCH = EW // C
NP = 10240

_SC_MESH = plsc.VectorSubcoreMesh(core_axis_name="c", subcore_axis_name="s")


def _gather_body(x_hbm, row3, col3, xi_hbm, xj_hbm, idxr, idxc, bufa, bufb, sem):
    cid = lax.axis_index("c")
    sid = lax.axis_index("s")
    wid = cid * NS + sid
    base = wid * EW
    pltpu.sync_copy(row3.at[wid], idxr)
    pltpu.sync_copy(col3.at[wid], idxc)

    def chunk(j, carry):
        ca = pltpu.async_copy(x_hbm.at[idxr.at[j]], bufa, sem)
        cb = pltpu.async_copy(x_hbm.at[idxc.at[j]], bufb, sem)
        ca.wait()
        cb.wait()
        pltpu.sync_copy(bufa, xi_hbm.at[pl.ds(base + j * C, C)])
        pltpu.sync_copy(bufb, xj_hbm.at[pl.ds(base + j * C, C)])
        return carry

    lax.fori_loop(0, NCH, chunk, 0)


_gather = pl.kernel(
    _gather_body,
    out_type=(
        jax.ShapeDtypeStruct((E, D), jnp.float32),
        jax.ShapeDtypeStruct((E, D), jnp.float32),
    ),
    mesh=_SC_MESH,
    scratch_types=[
        pltpu.VMEM((NCH, C), jnp.int32),
        pltpu.VMEM((NCH, C), jnp.int32),
        pltpu.VMEM((C, D), jnp.float32),
        pltpu.VMEM((C, D), jnp.float32),
        pltpu.SemaphoreType.DMA,
    ],
)


def _scatter_body(emb_hbm, row3, zeros_hbm, out_hbm, idxr, ebuf, agg_sh, sem):
    cid = lax.axis_index("c")
    sid = lax.axis_index("s")
    wid = cid * NS + sid
    base = wid * EW
    rps = NP // NS
    pltpu.sync_copy(zeros_hbm.at[pl.ds(sid * rps, rps)],
                    agg_sh.at[pl.ds(sid * rps, rps)])
    pltpu.sync_copy(row3.at[wid], idxr)
    plsc.subcore_barrier()

    def chunk(j, carry):
        pltpu.sync_copy(emb_hbm.at[pl.ds(base + j * C, C)], ebuf)
        pltpu.sync_copy(ebuf, agg_sh.at[idxr.at[j]], add=True)
        return carry

    lax.fori_loop(0, NCH, chunk, 0)
    plsc.subcore_barrier()
    pltpu.sync_copy(agg_sh.at[pl.ds(sid * rps, rps)],
                    out_hbm.at[cid, pl.ds(sid * rps, rps)])


_scatter = pl.kernel(
    _scatter_body,
    out_type=jax.ShapeDtypeStruct((NC, NP, H), jnp.float32),
    mesh=_SC_MESH,
    scratch_types=[
        pltpu.VMEM((NCH, C), jnp.int32),
        pltpu.VMEM((C, H), jnp.float32),
        pltpu.VMEM_SHARED((NP, H), jnp.float32),
        pltpu.SemaphoreType.DMA,
    ],
)


BE = 1280

def _silu(v):
    return v / (1.0 + jnp.exp(-v))


def _bf(v):
    return v.astype(jnp.bfloat16)


def _dot(a, b):
    return jnp.dot(_bf(a), _bf(b), preferred_element_type=jnp.float32)


def _edge_body(xi, xj, ea, te,
               em_w1a, em_w1b, em_w1c, em_w1t, em_b1, em_w2, em_b2, em_w3, em_b3,
               ee_w1a, ee_w1b, ee_w1c, ee_b1, ee_w2, ee_b2, ee_w3, ee_b3,
               ean_out, emb_out):
    x1 = xi[:] + xj[:]
    x2 = jnp.abs(xi[:] - xj[:])
    ea_v = ea[:]
    pre = (_dot(x1, em_w1a[:]) + _dot(x2, em_w1b[:]) + _dot(ea_v, em_w1c[:])
           + te[:] * em_w1t[:] + em_b1[:])
    h = _silu(pre)
    h = _silu(_dot(h, em_w2[:]) + em_b2[:])
    ean = _dot(h, em_w3[:]) + em_b3[:] + ea_v
    pre2 = (_dot(x1, ee_w1a[:]) + _dot(x2, ee_w1b[:]) + _dot(ean, ee_w1c[:])
            + ee_b1[:])
    g = _silu(pre2)
    g = _silu(_dot(g, ee_w2[:]) + ee_b2[:])
    ean_out[:] = ean
    emb_out[:] = _dot(g, ee_w3[:]) + ee_b3[:]


def _edge_mlp(xi, xj, ea, te, em, ee):
    nb = E // BE
    row_spec = lambda d: pl.BlockSpec((BE, d), lambda i: (i, 0))
    w_spec = lambda a: pl.BlockSpec(a.shape, lambda i: (0, 0))
    return pl.pallas_call(
        _edge_body,
        grid=(nb,),
        in_specs=[row_spec(D), row_spec(D), row_spec(ED), row_spec(1)]
                 + [w_spec(a) for a in em] + [w_spec(a) for a in ee],
        out_specs=(row_spec(ED), row_spec(H)),
        out_shape=(
            jax.ShapeDtypeStruct((E, ED), jnp.float32),
            jax.ShapeDtypeStruct((E, H), jnp.float32),
        ),
        compiler_params=pltpu.CompilerParams(
            dimension_semantics=("arbitrary",)),
    )(xi, xj, ea, te, *em, *ee)


BN = 1000

def _node_body(x, p0, p1, tn, w1x, w1a, w1t, b1, w2, b2, w3, b3, out):
    agg = (p0[:] + p1[:]) * 0.01
    pre = (_dot(x[:], w1x[:]) + _dot(agg, w1a[:]) + tn[:] * w1t[:] + b1[:])
    h = _silu(pre)
    h = _silu(_dot(h, w2[:]) + b2[:])
    out[:] = _dot(h, w3[:]) + b3[:]


def _node_mlp(x, p0, p1, tn, nm):
    nb = N // BN
    row_spec = lambda d: pl.BlockSpec((BN, d), lambda i: (i, 0))
    w_spec = lambda a: pl.BlockSpec(a.shape, lambda i: (0, 0))
    return pl.pallas_call(
        _node_body,
        grid=(nb,),
        in_specs=[row_spec(D), row_spec(H), row_spec(H), row_spec(1)]
                 + [w_spec(a) for a in nm],
        out_specs=row_spec(H),
        out_shape=jax.ShapeDtypeStruct((N, H), jnp.float32),
        compiler_params=pltpu.CompilerParams(
            dimension_semantics=("arbitrary",)),
    )(x, p0, p1, tn, *nm)


def kernel(x, t, edge_index, edge_attr, batch_size,
           em_w1, em_b1, em_w2, em_b2, em_w3, em_b3,
           ee_w1, ee_b1, ee_w2, ee_b2, ee_w3, ee_b3,
           nm_w1, nm_b1, nm_w2, nm_b2, nm_w3, nm_b3):
    bs = t.shape[0]
    row = edge_index[0]
    col = edge_index[1]
    row3 = row.reshape(W, NCH, C)
    col3 = col.reshape(W, NCH, C)

    xi, xj = _gather(x, row3, col3)

    te = jnp.repeat(t, E // bs).reshape(E, 1)
    em = (em_w1[:D], em_w1[D:2 * D], em_w1[2 * D:2 * D + ED],
          em_w1[2 * D + ED:], em_b1.reshape(1, H), em_w2, em_b2.reshape(1, H),
          em_w3, em_b3.reshape(1, ED))
    ee = (ee_w1[:D], ee_w1[D:2 * D], ee_w1[2 * D:],
          ee_b1.reshape(1, H), ee_w2, ee_b2.reshape(1, H),
          ee_w3, ee_b3.reshape(1, H))
    ean, emb = _edge_mlp(xi, xj, edge_attr, te, em, ee)

    partials = _scatter(emb, row3, jnp.zeros((NP, H), jnp.float32))

    tn = jnp.repeat(t, N // bs).reshape(N, 1)
    nm = (nm_w1[:D], nm_w1[D:D + H], nm_w1[D + H:],
          nm_b1.reshape(1, D), nm_w2, nm_b2.reshape(1, D),
          nm_w3, nm_b3.reshape(1, H))
    x_out = _node_mlp(x, partials[0, :N], partials[1, :N], tn, nm)
    return (x_out, ean)
